# baseline (speedup 1.0000x reference)
import functools

import jax
import jax.numpy as jnp
from jax import lax
from jax.experimental import pallas as pl
from jax.experimental.pallas import tpu as pltpu
from jax.experimental.pallas import tpu_sc as plsc

NC = 2
NS = 16
NW = NC * NS
ENT_D = 64
REL_ROW = 384
ECHUNK = 128
TBLK = 2048


def _transpose_body(src, dst):
    dst[...] = src[...].T


def _entity_major(ent_t):
    v = ent_t.shape[1]
    grid = (pl.cdiv(v, TBLK),)
    return pl.pallas_call(
        _transpose_body,
        grid=grid,
        in_specs=[pl.BlockSpec((ENT_D, TBLK), lambda i: (0, i))],
        out_specs=pl.BlockSpec((TBLK, ENT_D), lambda i: (i, 0)),
        out_shape=jax.ShapeDtypeStruct((v, ENT_D), jnp.float32),
    )(ent_t)


def _sc_gather(ent_tab, rel_tab, ent_idx, rel_idx):
    _, epw = ent_idx.shape
    _, jr, rpc = rel_idx.shape

    mesh = plsc.VectorSubcoreMesh(core_axis_name="c", subcore_axis_name="s")

    @functools.partial(
        pl.kernel,
        mesh=mesh,
        compiler_params=pltpu.CompilerParams(use_tc_tiling_on_sc=True),
        out_type=[
            jax.ShapeDtypeStruct((NW * epw, ENT_D), jnp.float32),
            jax.ShapeDtypeStruct((NW * jr * rpc, REL_ROW), jnp.float32),
        ],
        scratch_types=[
            pltpu.VMEM((epw,), jnp.int32),
            pltpu.VMEM((jr, rpc), jnp.int32),
            pltpu.VMEM((epw, ENT_D), jnp.float32),
            pltpu.VMEM((rpc, REL_ROW), jnp.float32),
            pltpu.SemaphoreType.DMA,
            pltpu.SemaphoreType.DMA,
        ],
    )
    def k(ent_hbm, rel_hbm, eidx_hbm, ridx_hbm, eout_hbm, rout_hbm,
          eidx_v, ridx_v, erows_v, rrows_v, esem, rsem):
        wid = lax.axis_index("s") * NC + lax.axis_index("c")
        pltpu.sync_copy(eidx_hbm.at[wid], eidx_v)
        pltpu.sync_copy(ridx_hbm.at[wid], ridx_v)

        def fire(g, carry):
            base = g * 16
            idx16 = eidx_v[pl.ds(base, 16)]
            for j in range(16):
                row = jnp.squeeze(lax.slice(idx16, (j,), (j + 1,)))
                pltpu.async_copy(ent_hbm.at[pl.ds(row, 1)],
                                 erows_v.at[pl.ds(base + j, 1)], esem)
            return carry

        for cstart in range(0, epw, ECHUNK):
            lax.fori_loop(cstart // 16, (cstart + ECHUNK) // 16, fire, 0)
            pltpu.make_async_copy(
                eout_hbm.at[pl.ds(0, ECHUNK)],
                erows_v.at[pl.ds(cstart, ECHUNK)], esem).wait()

        pltpu.sync_copy(erows_v, eout_hbm.at[pl.ds(wid * epw, epw)])

        for j in range(jr):
            pltpu.async_copy(rel_hbm.at[ridx_v.at[j]], rrows_v, rsem).wait()
            pltpu.sync_copy(
                rrows_v, rout_hbm.at[pl.ds((wid * jr + j) * rpc, rpc)])

    return k(ent_tab, rel_tab, ent_idx, rel_idx)


def _tc_body(ph, pt, nh, nt, pr, nr, out):
    def l2n(x):
        ss = jnp.sum(x * x, axis=-1, keepdims=True)
        return x * lax.rsqrt(jnp.maximum(ss, 1e-12))

    def neg_log_score(h_raw, t_raw, rw):
        h = l2n(h_raw[...])
        t = l2n(t_raw[...])
        r = rw[...]
        w = r[:, 4 * ENT_D:4 * ENT_D + 4]
        wn = w * lax.rsqrt(
            jnp.maximum(jnp.sum(w * w, axis=-1, keepdims=True), 1e-12))
        ssum = None
        for c in range(4):
            rc = l2n(r[:, c * ENT_D:(c + 1) * ENT_D])
            d = rc + h - t
            n2 = jnp.sum(d * d, axis=-1, keepdims=True)
            term = wn[:, c:c + 1] * jnp.exp(n2)
            ssum = term if ssum is None else ssum + term
        return -jnp.log(jnp.maximum(ssum, 1e-8))

    p = neg_log_score(ph, pt, pr)
    n = neg_log_score(nh, nt, nr)
    blk = jnp.sum(jnp.maximum(p - n + 1.0, 0.0))

    @pl.when(pl.program_id(0) == 0)
    def _():
        out[...] = jnp.zeros((1, 1), jnp.float32)

    out[...] = out[...] + blk


def _tc_loss(ph_e, pt_e, nh_e, nt_e, pr_e, nr_e, blk):
    b = ph_e.shape[0]
    grid = (b // blk,)
    ent_spec = pl.BlockSpec((blk, ENT_D), lambda i: (i, 0))
    rel_spec = pl.BlockSpec((blk, REL_ROW), lambda i: (i, 0))
    return pl.pallas_call(
        _tc_body,
        grid=grid,
        in_specs=[ent_spec, ent_spec, ent_spec, ent_spec, rel_spec, rel_spec],
        out_specs=pl.BlockSpec((1, 1), lambda i: (0, 0)),
        out_shape=jax.ShapeDtypeStruct((1, 1), jnp.float32),
    )(ph_e, pt_e, nh_e, nt_e, pr_e, nr_e)


def kernel(pos_h, pos_t, pos_r, neg_h, neg_t, neg_r,
           ent_embeddings, rel_embeddings, rel_weights):
    b = pos_h.shape[0]
    rel_total, clus, rel_d = rel_embeddings.shape

    ent_tab = _entity_major(ent_embeddings.T)

    rel_tab = jnp.concatenate(
        [rel_embeddings.reshape(rel_total, clus * rel_d),
         rel_weights,
         jnp.zeros((rel_total, REL_ROW - clus * rel_d - clus), jnp.float32)],
        axis=1)

    ent_idx = jnp.concatenate(
        [pos_h, pos_t, neg_h, neg_t], axis=0).astype(jnp.int32)
    rel_idx = jnp.concatenate([pos_r, neg_r], axis=0).astype(jnp.int32)
    epw = 4 * b // NW
    ent_idx = ent_idx.reshape(NW, epw)
    rel_idx = rel_idx.reshape(NW, (2 * b // NW) // 128, 128)

    ent_rows, rel_rows = _sc_gather(ent_tab, rel_tab, ent_idx, rel_idx)

    loss = _tc_loss(ent_rows[0:b], ent_rows[b:2 * b],
                    ent_rows[2 * b:3 * b], ent_rows[3 * b:4 * b],
                    rel_rows[0:b], rel_rows[b:2 * b], blk=2048)
    return loss[0, 0]


# speedup vs baseline: 1.3255x; 1.3255x over previous
import functools

import jax
import jax.numpy as jnp
from jax import lax
from jax.experimental import pallas as pl
from jax.experimental.pallas import tpu as pltpu
from jax.experimental.pallas import tpu_sc as plsc

NC = 2
NS = 16
NW = NC * NS
ENT_D = 64
REL_ROW = 384
TBLK = 2048
HBLK = 245
POFF = HBLK * TBLK


def _transpose_body(lo, hi, dst):
    dst[...] = jnp.concatenate([lo[...].T, hi[...].T], axis=1)


def _pack_entities(ent_t):
    v = ent_t.shape[1]
    last = pl.cdiv(v, TBLK) - 1
    return pl.pallas_call(
        _transpose_body,
        grid=(HBLK,),
        in_specs=[
            pl.BlockSpec((ENT_D, TBLK), lambda i: (0, i)),
            pl.BlockSpec((ENT_D, TBLK),
                         lambda i, lb=last: (0, jnp.minimum(i + HBLK, lb))),
        ],
        out_specs=pl.BlockSpec((TBLK, 2 * ENT_D), lambda i: (i, 0)),
        out_shape=jax.ShapeDtypeStruct((POFF, 2 * ENT_D), jnp.float32),
    )(ent_t, ent_t)


def _sc_gather(ent_pairs, rel_tab, ent_idx, rel_idx):
    _, je, epc = ent_idx.shape
    _, jr, rpc = rel_idx.shape

    mesh = plsc.VectorSubcoreMesh(core_axis_name="c", subcore_axis_name="s")

    @functools.partial(
        pl.kernel,
        mesh=mesh,
        compiler_params=pltpu.CompilerParams(use_tc_tiling_on_sc=True),
        out_type=[
            jax.ShapeDtypeStruct((NW * je, epc, 2 * ENT_D), jnp.float32),
            jax.ShapeDtypeStruct((NW * jr, rpc, REL_ROW), jnp.float32),
        ],
        scratch_types=[
            pltpu.VMEM((je, epc), jnp.int32),
            pltpu.VMEM((jr, rpc), jnp.int32),
            pltpu.VMEM((je, epc, 2 * ENT_D), jnp.float32),
            pltpu.VMEM((rpc, REL_ROW), jnp.float32),
            pltpu.SemaphoreType.DMA,
            pltpu.SemaphoreType.DMA,
        ],
    )
    def k(ent_hbm, rel_hbm, eidx_hbm, ridx_hbm, eout_hbm, rout_hbm,
          eidx_v, ridx_v, erows_v, rrows_v, esem, rsem):
        wid = lax.axis_index("s") * NC + lax.axis_index("c")
        pltpu.sync_copy(eidx_hbm.at[wid], eidx_v)
        pltpu.sync_copy(ridx_hbm.at[wid], ridx_v)

        copies = []
        for j in range(je):
            copies.append(pltpu.async_copy(
                ent_hbm.at[eidx_v.at[j]], erows_v.at[j], esem))
        for j in range(jr):
            pltpu.async_copy(rel_hbm.at[ridx_v.at[j]], rrows_v, rsem).wait()
            pltpu.sync_copy(rrows_v, rout_hbm.at[j + wid * jr])
        for c in copies:
            c.wait()
        pltpu.sync_copy(erows_v, eout_hbm.at[pl.ds(wid * je, je)])

    return k(ent_pairs, rel_tab, ent_idx, rel_idx)


def _tc_body(ph, pt, nh, nt, pp, pr, nr, out):
    def sel(pair, par):
        return pair[:, :ENT_D] + par * (pair[:, ENT_D:] - pair[:, :ENT_D])

    def l2n(x):
        ss = jnp.sum(x * x, axis=-1, keepdims=True)
        return x * lax.rsqrt(jnp.maximum(ss, 1e-12))

    def neg_log_score(h_raw, t_raw, rw):
        h = l2n(h_raw)
        t = l2n(t_raw)
        r = rw[...]
        w = r[:, 4 * ENT_D:4 * ENT_D + 4]
        wn = w * lax.rsqrt(
            jnp.maximum(jnp.sum(w * w, axis=-1, keepdims=True), 1e-12))
        ssum = None
        for c in range(4):
            rc = l2n(r[:, c * ENT_D:(c + 1) * ENT_D])
            d = rc + h - t
            n2 = jnp.sum(d * d, axis=-1, keepdims=True)
            term = wn[:, c:c + 1] * jnp.exp(n2)
            ssum = term if ssum is None else ssum + term
        return -jnp.log(jnp.maximum(ssum, 1e-8))

    par = pp[...]
    p = neg_log_score(sel(ph[...], par[:, 0:1]), sel(pt[...], par[:, 1:2]), pr)
    n = neg_log_score(sel(nh[...], par[:, 2:3]), sel(nt[...], par[:, 3:4]), nr)
    blk = jnp.sum(jnp.maximum(p - n + 1.0, 0.0))

    @pl.when(pl.program_id(0) == 0)
    def _():
        out[...] = jnp.zeros((1, 1), jnp.float32)

    out[...] = out[...] + blk


def _tc_loss(ph_e, pt_e, nh_e, nt_e, par, pr_e, nr_e, blk):
    b = ph_e.shape[0]
    grid = (b // blk,)
    ent_spec = pl.BlockSpec((blk, 2 * ENT_D), lambda i: (i, 0))
    par_spec = pl.BlockSpec((blk, 4), lambda i: (i, 0))
    rel_spec = pl.BlockSpec((blk, REL_ROW), lambda i: (i, 0))
    return pl.pallas_call(
        _tc_body,
        grid=grid,
        in_specs=[ent_spec, ent_spec, ent_spec, ent_spec, par_spec,
                  rel_spec, rel_spec],
        out_specs=pl.BlockSpec((1, 1), lambda i: (0, 0)),
        out_shape=jax.ShapeDtypeStruct((1, 1), jnp.float32),
    )(ph_e, pt_e, nh_e, nt_e, par, pr_e, nr_e)


def kernel(pos_h, pos_t, pos_r, neg_h, neg_t, neg_r,
           ent_embeddings, rel_embeddings, rel_weights):
    b = pos_h.shape[0]
    half = POFF
    rel_total, clus, rel_d = rel_embeddings.shape

    ent_pairs = _pack_entities(ent_embeddings.T)

    rel_tab = jnp.concatenate(
        [rel_embeddings.reshape(rel_total, clus * rel_d),
         rel_weights,
         jnp.zeros((rel_total, REL_ROW - clus * rel_d - clus), jnp.float32)],
        axis=1)

    ent_idx = jnp.concatenate(
        [pos_h, pos_t, neg_h, neg_t], axis=0).astype(jnp.int32)
    par = (jnp.concatenate([pos_h, pos_t, neg_h, neg_t], axis=1)
           >= half).astype(jnp.float32)
    rel_idx = jnp.concatenate([pos_r, neg_r], axis=0).astype(jnp.int32)
    epw = 4 * b // NW
    ent_idx = jnp.where(ent_idx < half, ent_idx, ent_idx - half)
    ent_idx = ent_idx.reshape(NW, epw // 128, 128)
    rel_idx = rel_idx.reshape(NW, (2 * b // NW) // 128, 128)

    ent_rows, rel_rows = _sc_gather(ent_pairs, rel_tab, ent_idx, rel_idx)
    ent_rows = ent_rows.reshape(4 * b, 2 * ENT_D)
    rel_rows = rel_rows.reshape(2 * b, REL_ROW)

    loss = _tc_loss(ent_rows[0:b], ent_rows[b:2 * b],
                    ent_rows[2 * b:3 * b], ent_rows[3 * b:4 * b],
                    par, rel_rows[0:b], rel_rows[b:2 * b], blk=2048)
    return loss[0, 0]


# speedup vs baseline: 1.5924x; 1.2014x over previous
import functools

import jax
import jax.numpy as jnp
from jax import lax
from jax.experimental import pallas as pl
from jax.experimental.pallas import tpu as pltpu
from jax.experimental.pallas import tpu_sc as plsc

NC = 2
NS = 16
NW = NC * NS
ENT_D = 64
REL_ROW = 384
TBLK = 4096
HBLK = 123
POFF = HBLK * TBLK


def _transpose_body(lo, hi, dst):
    dst[...] = jnp.concatenate([lo[...].T, hi[...].T], axis=1)


def _pack_entities(ent_t):
    v = ent_t.shape[1]
    last = pl.cdiv(v, TBLK) - 1
    return pl.pallas_call(
        _transpose_body,
        grid=(HBLK,),
        in_specs=[
            pl.BlockSpec((ENT_D, TBLK), lambda i: (0, i)),
            pl.BlockSpec((ENT_D, TBLK),
                         lambda i, lb=last: (0, jnp.minimum(i + HBLK, lb))),
        ],
        out_specs=pl.BlockSpec((TBLK, 2 * ENT_D), lambda i: (i, 0)),
        out_shape=jax.ShapeDtypeStruct((POFF, 2 * ENT_D), jnp.float32),
    )(ent_t, ent_t)


def _sc_gather(ent_pairs, rel_tab, ent_idx, rel_idx):
    _, je, epc = ent_idx.shape
    _, jr, rpc = rel_idx.shape

    mesh = plsc.VectorSubcoreMesh(core_axis_name="c", subcore_axis_name="s")

    @functools.partial(
        pl.kernel,
        mesh=mesh,
        compiler_params=pltpu.CompilerParams(use_tc_tiling_on_sc=True),
        out_type=[
            jax.ShapeDtypeStruct((NW * je, epc, 2 * ENT_D), jnp.float32),
            jax.ShapeDtypeStruct((NW * jr, rpc, REL_ROW), jnp.float32),
        ],
        scratch_types=[
            pltpu.VMEM((je, epc), jnp.int32),
            pltpu.VMEM((jr, rpc), jnp.int32),
            pltpu.VMEM((je, epc, 2 * ENT_D), jnp.float32),
            pltpu.VMEM((rpc, REL_ROW), jnp.float32),
            pltpu.SemaphoreType.DMA,
            pltpu.SemaphoreType.DMA,
        ],
    )
    def k(ent_hbm, rel_hbm, eidx_hbm, ridx_hbm, eout_hbm, rout_hbm,
          eidx_v, ridx_v, erows_v, rrows_v, esem, rsem):
        wid = lax.axis_index("s") * NC + lax.axis_index("c")
        pltpu.sync_copy(eidx_hbm.at[wid], eidx_v)
        pltpu.sync_copy(ridx_hbm.at[wid], ridx_v)

        copies = []
        for j in range(je):
            copies.append(pltpu.async_copy(
                ent_hbm.at[eidx_v.at[j]], erows_v.at[j], esem))
        for j in range(jr):
            pltpu.async_copy(rel_hbm.at[ridx_v.at[j]], rrows_v, rsem).wait()
            pltpu.sync_copy(rrows_v, rout_hbm.at[j + wid * jr])
        for c in copies:
            c.wait()
        pltpu.sync_copy(erows_v, eout_hbm.at[pl.ds(wid * je, je)])

    return k(ent_pairs, rel_tab, ent_idx, rel_idx)


def _tc_body(ph, pt, nh, nt, pp, pr, nr, out):
    def sel(pair, par):
        return pair[:, :ENT_D] + par * (pair[:, ENT_D:] - pair[:, :ENT_D])

    def l2n(x):
        ss = jnp.sum(x * x, axis=-1, keepdims=True)
        return x * lax.rsqrt(jnp.maximum(ss, 1e-12))

    def neg_log_score(h_raw, t_raw, rw):
        h = l2n(h_raw)
        t = l2n(t_raw)
        r = rw[...]
        w = r[:, 4 * ENT_D:4 * ENT_D + 4]
        wn = w * lax.rsqrt(
            jnp.maximum(jnp.sum(w * w, axis=-1, keepdims=True), 1e-12))
        ssum = None
        for c in range(4):
            rc = l2n(r[:, c * ENT_D:(c + 1) * ENT_D])
            d = rc + h - t
            n2 = jnp.sum(d * d, axis=-1, keepdims=True)
            term = wn[:, c:c + 1] * jnp.exp(n2)
            ssum = term if ssum is None else ssum + term
        return -jnp.log(jnp.maximum(ssum, 1e-8))

    par = pp[...]
    p = neg_log_score(sel(ph[...], par[:, 0:1]), sel(pt[...], par[:, 1:2]), pr)
    n = neg_log_score(sel(nh[...], par[:, 2:3]), sel(nt[...], par[:, 3:4]), nr)
    blk = jnp.sum(jnp.maximum(p - n + 1.0, 0.0))

    @pl.when(pl.program_id(0) == 0)
    def _():
        out[...] = jnp.zeros((1, 1), jnp.float32)

    out[...] = out[...] + blk


def _tc_loss(ph_e, pt_e, nh_e, nt_e, par, pr_e, nr_e, blk):
    b = ph_e.shape[0]
    grid = (b // blk,)
    ent_spec = pl.BlockSpec((blk, 2 * ENT_D), lambda i: (i, 0))
    par_spec = pl.BlockSpec((blk, 4), lambda i: (i, 0))
    rel_spec = pl.BlockSpec((blk, REL_ROW), lambda i: (i, 0))
    return pl.pallas_call(
        _tc_body,
        grid=grid,
        in_specs=[ent_spec, ent_spec, ent_spec, ent_spec, par_spec,
                  rel_spec, rel_spec],
        out_specs=pl.BlockSpec((1, 1), lambda i: (0, 0)),
        out_shape=jax.ShapeDtypeStruct((1, 1), jnp.float32),
    )(ph_e, pt_e, nh_e, nt_e, par, pr_e, nr_e)


def kernel(pos_h, pos_t, pos_r, neg_h, neg_t, neg_r,
           ent_embeddings, rel_embeddings, rel_weights):
    b = pos_h.shape[0]
    half = POFF
    rel_total, clus, rel_d = rel_embeddings.shape

    ent_pairs = _pack_entities(ent_embeddings.T)

    rel_tab = jnp.concatenate(
        [rel_embeddings.reshape(rel_total, clus * rel_d),
         rel_weights,
         jnp.zeros((rel_total, REL_ROW - clus * rel_d - clus), jnp.float32)],
        axis=1)

    ent_idx = jnp.concatenate(
        [pos_h, pos_t, neg_h, neg_t], axis=0).astype(jnp.int32)
    par = (jnp.concatenate([pos_h, pos_t, neg_h, neg_t], axis=1)
           >= half).astype(jnp.float32)
    rel_idx = jnp.concatenate([pos_r, neg_r], axis=0).astype(jnp.int32)
    epw = 4 * b // NW
    ent_idx = jnp.where(ent_idx < half, ent_idx, ent_idx - half)
    ent_idx = ent_idx.reshape(NW, epw // 128, 128)
    rel_idx = rel_idx.reshape(NW, (2 * b // NW) // 128, 128)

    ent_rows, rel_rows = _sc_gather(ent_pairs, rel_tab, ent_idx, rel_idx)
    ent_rows = ent_rows.reshape(4 * b, 2 * ENT_D)
    rel_rows = rel_rows.reshape(2 * b, REL_ROW)

    loss = _tc_loss(ent_rows[0:b], ent_rows[b:2 * b],
                    ent_rows[2 * b:3 * b], ent_rows[3 * b:4 * b],
                    par, rel_rows[0:b], rel_rows[b:2 * b], blk=2048)
    return loss[0, 0]


# speedup vs baseline: 1.7522x; 1.1004x over previous
import functools

import jax
import jax.numpy as jnp
from jax import lax
from jax.experimental import pallas as pl
from jax.experimental.pallas import tpu as pltpu
from jax.experimental.pallas import tpu_sc as plsc

NC = 2
NS = 16
NW = NC * NS
ENT_D = 64
REL_ROW = 384
TBLK = 8192
HBLK = 62
POFF = HBLK * TBLK


def _transpose_body(lo, hi, dst):
    dst[...] = jnp.concatenate([lo[...].T, hi[...].T], axis=1)


def _pack_entities(ent_t):
    v = ent_t.shape[1]
    last = pl.cdiv(v, TBLK) - 1
    return pl.pallas_call(
        _transpose_body,
        grid=(HBLK,),
        in_specs=[
            pl.BlockSpec((ENT_D, TBLK), lambda i: (0, i)),
            pl.BlockSpec((ENT_D, TBLK),
                         lambda i, lb=last: (0, jnp.minimum(i + HBLK, lb))),
        ],
        out_specs=pl.BlockSpec((TBLK, 2 * ENT_D), lambda i: (i, 0)),
        out_shape=jax.ShapeDtypeStruct((POFF, 2 * ENT_D), jnp.float32),
    )(ent_t, ent_t)


def _sc_gather(ent_pairs, rel_tab, ent_idx, rel_idx):
    _, je, epc = ent_idx.shape
    _, jr, rpc = rel_idx.shape

    mesh = plsc.VectorSubcoreMesh(core_axis_name="c", subcore_axis_name="s")

    @functools.partial(
        pl.kernel,
        mesh=mesh,
        compiler_params=pltpu.CompilerParams(use_tc_tiling_on_sc=True),
        out_type=[
            jax.ShapeDtypeStruct((NW * je, epc, 2 * ENT_D), jnp.float32),
            jax.ShapeDtypeStruct((NW * jr, rpc, REL_ROW), jnp.float32),
        ],
        scratch_types=[
            pltpu.VMEM((je, epc), jnp.int32),
            pltpu.VMEM((jr, rpc), jnp.int32),
            pltpu.VMEM((je, epc, 2 * ENT_D), jnp.float32),
            pltpu.VMEM((rpc, REL_ROW), jnp.float32),
            pltpu.SemaphoreType.DMA,
            pltpu.SemaphoreType.DMA,
        ],
    )
    def k(ent_hbm, rel_hbm, eidx_hbm, ridx_hbm, eout_hbm, rout_hbm,
          eidx_v, ridx_v, erows_v, rrows_v, esem, rsem):
        wid = lax.axis_index("s") * NC + lax.axis_index("c")
        pltpu.sync_copy(eidx_hbm.at[wid], eidx_v)
        pltpu.sync_copy(ridx_hbm.at[wid], ridx_v)

        copies = []
        for j in range(je):
            copies.append(pltpu.async_copy(
                ent_hbm.at[eidx_v.at[j]], erows_v.at[j], esem))
        for j in range(jr):
            pltpu.async_copy(rel_hbm.at[ridx_v.at[j]], rrows_v, rsem).wait()
            pltpu.sync_copy(rrows_v, rout_hbm.at[j + wid * jr])
        for c in copies:
            c.wait()
        pltpu.sync_copy(erows_v, eout_hbm.at[pl.ds(wid * je, je)])

    return k(ent_pairs, rel_tab, ent_idx, rel_idx)


def _tc_body(ph, pt, nh, nt, pp, pr, nr, out):
    def sel(pair, par):
        return pair[:, :ENT_D] + par * (pair[:, ENT_D:] - pair[:, :ENT_D])

    def l2n(x):
        ss = jnp.sum(x * x, axis=-1, keepdims=True)
        return x * lax.rsqrt(jnp.maximum(ss, 1e-12))

    def neg_log_score(h_raw, t_raw, rw):
        h = l2n(h_raw)
        t = l2n(t_raw)
        r = rw[...]
        w = r[:, 4 * ENT_D:4 * ENT_D + 4]
        wn = w * lax.rsqrt(
            jnp.maximum(jnp.sum(w * w, axis=-1, keepdims=True), 1e-12))
        ssum = None
        for c in range(4):
            rc = l2n(r[:, c * ENT_D:(c + 1) * ENT_D])
            d = rc + h - t
            n2 = jnp.sum(d * d, axis=-1, keepdims=True)
            term = wn[:, c:c + 1] * jnp.exp(n2)
            ssum = term if ssum is None else ssum + term
        return -jnp.log(jnp.maximum(ssum, 1e-8))

    par = pp[...]
    p = neg_log_score(sel(ph[...], par[:, 0:1]), sel(pt[...], par[:, 1:2]), pr)
    n = neg_log_score(sel(nh[...], par[:, 2:3]), sel(nt[...], par[:, 3:4]), nr)
    blk = jnp.sum(jnp.maximum(p - n + 1.0, 0.0))

    @pl.when(pl.program_id(0) == 0)
    def _():
        out[...] = jnp.zeros((1, 1), jnp.float32)

    out[...] = out[...] + blk


def _tc_loss(ph_e, pt_e, nh_e, nt_e, par, pr_e, nr_e, blk):
    b = ph_e.shape[0]
    grid = (b // blk,)
    ent_spec = pl.BlockSpec((blk, 2 * ENT_D), lambda i: (i, 0))
    par_spec = pl.BlockSpec((blk, 4), lambda i: (i, 0))
    rel_spec = pl.BlockSpec((blk, REL_ROW), lambda i: (i, 0))
    return pl.pallas_call(
        _tc_body,
        grid=grid,
        in_specs=[ent_spec, ent_spec, ent_spec, ent_spec, par_spec,
                  rel_spec, rel_spec],
        out_specs=pl.BlockSpec((1, 1), lambda i: (0, 0)),
        out_shape=jax.ShapeDtypeStruct((1, 1), jnp.float32),
    )(ph_e, pt_e, nh_e, nt_e, par, pr_e, nr_e)


def kernel(pos_h, pos_t, pos_r, neg_h, neg_t, neg_r,
           ent_embeddings, rel_embeddings, rel_weights):
    b = pos_h.shape[0]
    half = POFF
    rel_total, clus, rel_d = rel_embeddings.shape

    ent_pairs = _pack_entities(ent_embeddings.T)

    rel_tab = jnp.concatenate(
        [rel_embeddings.reshape(rel_total, clus * rel_d),
         rel_weights,
         jnp.zeros((rel_total, REL_ROW - clus * rel_d - clus), jnp.float32)],
        axis=1)

    ent_idx = jnp.concatenate(
        [pos_h, pos_t, neg_h, neg_t], axis=0).astype(jnp.int32)
    par = (jnp.concatenate([pos_h, pos_t, neg_h, neg_t], axis=1)
           >= half).astype(jnp.float32)
    rel_idx = jnp.concatenate([pos_r, neg_r], axis=0).astype(jnp.int32)
    epw = 4 * b // NW
    ent_idx = jnp.where(ent_idx < half, ent_idx, ent_idx - half)
    ent_idx = ent_idx.reshape(NW, epw // 128, 128)
    rel_idx = rel_idx.reshape(NW, (2 * b // NW) // 128, 128)

    ent_rows, rel_rows = _sc_gather(ent_pairs, rel_tab, ent_idx, rel_idx)
    ent_rows = ent_rows.reshape(4 * b, 2 * ENT_D)
    rel_rows = rel_rows.reshape(2 * b, REL_ROW)

    loss = _tc_loss(ent_rows[0:b], ent_rows[b:2 * b],
                    ent_rows[2 * b:3 * b], ent_rows[3 * b:4 * b],
                    par, rel_rows[0:b], rel_rows[b:2 * b], blk=2048)
    return loss[0, 0]


# speedup vs baseline: 1.8384x; 1.0492x over previous
import functools

import jax
import jax.numpy as jnp
from jax import lax
from jax.experimental import pallas as pl
from jax.experimental.pallas import tpu as pltpu
from jax.experimental.pallas import tpu_sc as plsc

NC = 2
NS = 16
NW = NC * NS
ENT_D = 64
REL_ROW = 384
TBLK = 16384
HBLK = 31
POFF = HBLK * TBLK


def _transpose_body(lo, hi, dst):
    dst[...] = jnp.concatenate([lo[...].T, hi[...].T], axis=1)


def _pack_entities(ent_t):
    v = ent_t.shape[1]
    last = pl.cdiv(v, TBLK) - 1
    return pl.pallas_call(
        _transpose_body,
        grid=(HBLK,),
        in_specs=[
            pl.BlockSpec((ENT_D, TBLK), lambda i: (0, i)),
            pl.BlockSpec((ENT_D, TBLK),
                         lambda i, lb=last: (0, jnp.minimum(i + HBLK, lb))),
        ],
        out_specs=pl.BlockSpec((TBLK, 2 * ENT_D), lambda i: (i, 0)),
        out_shape=jax.ShapeDtypeStruct((POFF, 2 * ENT_D), jnp.float32),
    )(ent_t, ent_t)


def _sc_gather(ent_pairs, rel_tab, ent_idx, rel_idx):
    _, je, epc = ent_idx.shape
    _, jr, rpc = rel_idx.shape

    mesh = plsc.VectorSubcoreMesh(core_axis_name="c", subcore_axis_name="s")

    @functools.partial(
        pl.kernel,
        mesh=mesh,
        compiler_params=pltpu.CompilerParams(use_tc_tiling_on_sc=True),
        out_type=[
            jax.ShapeDtypeStruct((NW * je, epc, 2 * ENT_D), jnp.float32),
            jax.ShapeDtypeStruct((NW * jr, rpc, REL_ROW), jnp.float32),
        ],
        scratch_types=[
            pltpu.VMEM((je, epc), jnp.int32),
            pltpu.VMEM((jr, rpc), jnp.int32),
            pltpu.VMEM((je, epc, 2 * ENT_D), jnp.float32),
            pltpu.VMEM((rpc, REL_ROW), jnp.float32),
            pltpu.SemaphoreType.DMA,
            pltpu.SemaphoreType.DMA,
        ],
    )
    def k(ent_hbm, rel_hbm, eidx_hbm, ridx_hbm, eout_hbm, rout_hbm,
          eidx_v, ridx_v, erows_v, rrows_v, esem, rsem):
        wid = lax.axis_index("s") * NC + lax.axis_index("c")
        pltpu.sync_copy(eidx_hbm.at[wid], eidx_v)
        pltpu.sync_copy(ridx_hbm.at[wid], ridx_v)

        copies = []
        for j in range(je):
            copies.append(pltpu.async_copy(
                ent_hbm.at[eidx_v.at[j]], erows_v.at[j], esem))
        for j in range(jr):
            pltpu.async_copy(rel_hbm.at[ridx_v.at[j]], rrows_v, rsem).wait()
            pltpu.sync_copy(rrows_v, rout_hbm.at[j + wid * jr])
        for c in copies:
            c.wait()
        pltpu.sync_copy(erows_v, eout_hbm.at[pl.ds(wid * je, je)])

    return k(ent_pairs, rel_tab, ent_idx, rel_idx)


def _tc_body(ph, pt, nh, nt, pp, pr, nr, out):
    def sel(pair, par):
        return pair[:, :ENT_D] + par * (pair[:, ENT_D:] - pair[:, :ENT_D])

    def l2n(x):
        ss = jnp.sum(x * x, axis=-1, keepdims=True)
        return x * lax.rsqrt(jnp.maximum(ss, 1e-12))

    def neg_log_score(h_raw, t_raw, rw):
        h = l2n(h_raw)
        t = l2n(t_raw)
        r = rw[...]
        w = r[:, 4 * ENT_D:4 * ENT_D + 4]
        wn = w * lax.rsqrt(
            jnp.maximum(jnp.sum(w * w, axis=-1, keepdims=True), 1e-12))
        ssum = None
        for c in range(4):
            rc = l2n(r[:, c * ENT_D:(c + 1) * ENT_D])
            d = rc + h - t
            n2 = jnp.sum(d * d, axis=-1, keepdims=True)
            term = wn[:, c:c + 1] * jnp.exp(n2)
            ssum = term if ssum is None else ssum + term
        return -jnp.log(jnp.maximum(ssum, 1e-8))

    par = pp[...]
    p = neg_log_score(sel(ph[...], par[:, 0:1]), sel(pt[...], par[:, 1:2]), pr)
    n = neg_log_score(sel(nh[...], par[:, 2:3]), sel(nt[...], par[:, 3:4]), nr)
    blk = jnp.sum(jnp.maximum(p - n + 1.0, 0.0))

    @pl.when(pl.program_id(0) == 0)
    def _():
        out[...] = jnp.zeros((1, 1), jnp.float32)

    out[...] = out[...] + blk


def _tc_loss(ph_e, pt_e, nh_e, nt_e, par, pr_e, nr_e, blk):
    b = ph_e.shape[0]
    grid = (b // blk,)
    ent_spec = pl.BlockSpec((blk, 2 * ENT_D), lambda i: (i, 0))
    par_spec = pl.BlockSpec((blk, 4), lambda i: (i, 0))
    rel_spec = pl.BlockSpec((blk, REL_ROW), lambda i: (i, 0))
    return pl.pallas_call(
        _tc_body,
        grid=grid,
        in_specs=[ent_spec, ent_spec, ent_spec, ent_spec, par_spec,
                  rel_spec, rel_spec],
        out_specs=pl.BlockSpec((1, 1), lambda i: (0, 0)),
        out_shape=jax.ShapeDtypeStruct((1, 1), jnp.float32),
    )(ph_e, pt_e, nh_e, nt_e, par, pr_e, nr_e)


def kernel(pos_h, pos_t, pos_r, neg_h, neg_t, neg_r,
           ent_embeddings, rel_embeddings, rel_weights):
    b = pos_h.shape[0]
    half = POFF
    rel_total, clus, rel_d = rel_embeddings.shape

    ent_pairs = _pack_entities(ent_embeddings.T)

    rel_tab = jnp.concatenate(
        [rel_embeddings.reshape(rel_total, clus * rel_d),
         rel_weights,
         jnp.zeros((rel_total, REL_ROW - clus * rel_d - clus), jnp.float32)],
        axis=1)

    ent_idx = jnp.concatenate(
        [pos_h, pos_t, neg_h, neg_t], axis=0).astype(jnp.int32)
    par = (jnp.concatenate([pos_h, pos_t, neg_h, neg_t], axis=1)
           >= half).astype(jnp.float32)
    rel_idx = jnp.concatenate([pos_r, neg_r], axis=0).astype(jnp.int32)
    epw = 4 * b // NW
    ent_idx = jnp.where(ent_idx < half, ent_idx, ent_idx - half)
    ent_idx = ent_idx.reshape(NW, epw // 128, 128)
    rel_idx = rel_idx.reshape(NW, (2 * b // NW) // 128, 128)

    ent_rows, rel_rows = _sc_gather(ent_pairs, rel_tab, ent_idx, rel_idx)
    ent_rows = ent_rows.reshape(4 * b, 2 * ENT_D)
    rel_rows = rel_rows.reshape(2 * b, REL_ROW)

    loss = _tc_loss(ent_rows[0:b], ent_rows[b:2 * b],
                    ent_rows[2 * b:3 * b], ent_rows[3 * b:4 * b],
                    par, rel_rows[0:b], rel_rows[b:2 * b], blk=2048)
    return loss[0, 0]
